# baseline (speedup 1.0000x reference)
import functools

import jax
import jax.numpy as jnp
from jax import lax
from jax.experimental import pallas as pl
from jax.experimental.pallas import tpu as pltpu
from jax.experimental.pallas import tpu_sc as plsc

D_MODEL = 64
NBUF = 4
T_BLOCK = 2048


def _make_transpose_body(v):
    n_grid = (v + T_BLOCK - 1) // T_BLOCK
    last = n_grid - 1
    last_rows = v - last * T_BLOCK

    def body(tt_ref, out_ref, s0, s1, sem0, sem1):
        i = pl.program_id(0)

        def step(s, sem, other, osem):
            @pl.when(i >= 2)
            def _():
                pltpu.make_async_copy(
                    s, out_ref.at[pl.ds((i - 2) * T_BLOCK, T_BLOCK), 0], sem
                ).wait()

            s[...] = tt_ref[...].T

            @pl.when(i < last)
            def _():
                pltpu.async_copy(
                    s, out_ref.at[pl.ds(i * T_BLOCK, T_BLOCK), 0], sem
                )

            @pl.when(i == last)
            def _():
                pltpu.make_async_copy(
                    other, out_ref.at[pl.ds((i - 1) * T_BLOCK, T_BLOCK), 0], osem
                ).wait()
                pltpu.async_copy(
                    s.at[pl.ds(0, last_rows)],
                    out_ref.at[pl.ds(i * T_BLOCK, last_rows), 0],
                    sem,
                ).wait()

        @pl.when(i % 2 == 0)
        def _():
            step(s0, sem0, s1, sem1)

        @pl.when(i % 2 == 1)
        def _():
            step(s1, sem1, s0, sem0)

    return body, n_grid


@jax.jit
def _tc_transpose(tt):
    d, v = tt.shape
    body, n_grid = _make_transpose_body(v)
    return pl.pallas_call(
        body,
        grid=(n_grid,),
        in_specs=[pl.BlockSpec((d, T_BLOCK), lambda i: (0, i))],
        out_specs=pl.BlockSpec(memory_space=pl.ANY),
        out_shape=jax.ShapeDtypeStruct((v, 2, d), jnp.float32),
        scratch_shapes=[
            pltpu.VMEM((T_BLOCK, d), jnp.float32),
            pltpu.VMEM((T_BLOCK, d), jnp.float32),
            pltpu.SemaphoreType.DMA,
            pltpu.SemaphoreType.DMA,
        ],
        compiler_params=pltpu.CompilerParams(
            dimension_semantics=("arbitrary",),
        ),
    )(tt)


@jax.jit
def _sc_embedding_lookup_t(xt2, table_padded_rows):
    seq_len, n_seq = xt2.shape
    info = plsc.get_sparse_core_info()
    nc, ns, nl = info.num_cores, info.num_subcores, info.num_lanes
    nw = nc * ns
    b_per_w = n_seq // nw

    mesh = plsc.VectorSubcoreMesh(core_axis_name="c", subcore_axis_name="s")

    @functools.partial(
        pl.kernel,
        mesh=mesh,
        out_type=jax.ShapeDtypeStruct(
            (seq_len, D_MODEL // 8, n_seq // 128, 8, 128), jnp.float32
        ),
        scratch_types=[
            pltpu.VMEM((seq_len, b_per_w), jnp.int32),
            pltpu.VMEM((NBUF, b_per_w, D_MODEL), jnp.float32),
            pltpu.VMEM((D_MODEL // 8, 8, b_per_w), jnp.float32),
            [pltpu.SemaphoreType.DMA] * NBUF,
            pltpu.SemaphoreType.DMA,
        ],
        compiler_params=pltpu.CompilerParams(
            use_tc_tiling_on_sc=False, needs_layout_passes=False
        ),
    )
    def k(x_hbm, table_hbm, out_hbm, idx_v, rows_v, t_v, sems, osem):
        wid = lax.axis_index("s") * nc + lax.axis_index("c")
        b0 = wid * b_per_w
        pltpu.sync_copy(x_hbm.at[:, pl.ds(b0, b_per_w)], idx_v)

        def gather(s, b):
            pltpu.async_copy(table_hbm.at[idx_v.at[s]], rows_v.at[b], sems[b])

        def wait_gather(s, b):
            pltpu.make_async_copy(
                table_hbm.at[idx_v.at[s]], rows_v.at[b], sems[b]
            ).wait()

        lanes = lax.iota(jnp.int32, nl)

        def transpose_rows(b):
            g = rows_v.at[b]

            @plsc.parallel_loop(0, D_MODEL, step=1, unroll=8)
            def d_body(d):
                dvec = jnp.full((nl,), d, jnp.int32)
                for kk in range(b_per_w // nl):
                    bvec = lanes + kk * nl
                    v = plsc.load_gather(g, [bvec, dvec])
                    t_v[d // 8, d % 8, pl.ds(kk * nl, nl)] = v

        def write_out(s):
            pltpu.async_copy(t_v, out_hbm.at[s, :, wid], osem).wait()

        for b in range(NBUF):
            gather(b, b)

        def ring_body(t, carry):
            s0 = t * NBUF
            for b in range(NBUF):
                s = s0 + b
                wait_gather(s, b)
                transpose_rows(b)
                write_out(s)
                gather(s + NBUF, b)
            return carry

        n_full = seq_len // NBUF - 1
        lax.fori_loop(0, n_full, ring_body, 0)

        done = n_full * NBUF
        for s in range(done, seq_len):
            b = s % NBUF
            wait_gather(s, b)
            transpose_rows(b)
            write_out(s)
            if s + NBUF < seq_len:
                gather(s + NBUF, b)

    return k(xt2, table_padded_rows)


def kernel(x, table):
    tbl = _tc_transpose(table.T).reshape(2 * table.shape[0], D_MODEL)
    m5 = _sc_embedding_lookup_t((x * 2).T, tbl)
    m = jnp.transpose(m5, (0, 1, 3, 2, 4)).reshape(x.shape[1], D_MODEL, x.shape[0])
    return jnp.transpose(m, (2, 0, 1))


# speedup vs baseline: 2.1839x; 2.1839x over previous
import functools

import jax
import jax.numpy as jnp
from jax import lax
from jax.experimental import pallas as pl
from jax.experimental.pallas import tpu as pltpu
from jax.experimental.pallas import tpu_sc as plsc

D_MODEL = 64
NBUF = 8
T_BLOCK = 2048


def _transpose_block(tt_ref, out_ref):
    out_ref[:, 0:64] = tt_ref[...].T


@jax.jit
def _tc_transpose(tt):
    d, v = tt.shape
    grid = (v + T_BLOCK - 1) // T_BLOCK
    return pl.pallas_call(
        _transpose_block,
        grid=(grid,),
        in_specs=[pl.BlockSpec((d, T_BLOCK), lambda i: (0, i))],
        out_specs=pl.BlockSpec((T_BLOCK, 2 * d), lambda i: (i, 0)),
        out_shape=jax.ShapeDtypeStruct((v, 2 * d), jnp.float32),
        compiler_params=pltpu.CompilerParams(
            dimension_semantics=("arbitrary",),
        ),
    )(tt)


@jax.jit
def _sc_embedding_lookup(x2, table_padded_rows):
    n_seq, seq_len = x2.shape
    info = plsc.get_sparse_core_info()
    nc, ns = info.num_cores, info.num_subcores
    nw = nc * ns
    seq_per_w = n_seq // nw
    assert seq_per_w % NBUF == 0

    mesh = plsc.VectorSubcoreMesh(core_axis_name="c", subcore_axis_name="s")

    @functools.partial(
        pl.kernel,
        mesh=mesh,
        out_type=jax.ShapeDtypeStruct((n_seq, seq_len, D_MODEL), jnp.float32),
        scratch_types=[
            pltpu.VMEM((seq_per_w, seq_len), jnp.int32),
            pltpu.VMEM((NBUF, seq_len, D_MODEL), jnp.float32),
            [pltpu.SemaphoreType.DMA] * NBUF,
        ],
        compiler_params=pltpu.CompilerParams(use_tc_tiling_on_sc=False),
    )
    def k(x_hbm, table_hbm, out_hbm, idx_v, rows_v, sems):
        wid = lax.axis_index("s") * nc + lax.axis_index("c")
        seq_base = wid * seq_per_w
        pltpu.sync_copy(x_hbm.at[pl.ds(seq_base, seq_per_w)], idx_v)

        def gather(j, b):
            pltpu.async_copy(table_hbm.at[idx_v.at[j]], rows_v.at[b], sems[b])

        def wait_gather(j, b):
            pltpu.make_async_copy(
                table_hbm.at[idx_v.at[j]], rows_v.at[b], sems[b]
            ).wait()

        def write_out(j, b):
            pltpu.sync_copy(rows_v.at[b], out_hbm.at[seq_base + j])

        for b in range(NBUF):
            gather(b, b)

        def ring_body(t, carry):
            j0 = t * NBUF
            for b in range(NBUF):
                j = j0 + b
                wait_gather(j, b)
                write_out(j, b)
                gather(j + NBUF, b)
            return carry

        lax.fori_loop(0, seq_per_w // NBUF - 1, ring_body, 0)

        j0 = seq_per_w - NBUF
        for b in range(NBUF):
            j = j0 + b
            wait_gather(j, b)
            write_out(j, b)

    return k(x2, table_padded_rows)


def kernel(x, table):
    tbl128 = _tc_transpose(table.T)
    tbl = tbl128.reshape(2 * table.shape[0], D_MODEL)
    return _sc_embedding_lookup(x * 2, tbl)
